# baseline (speedup 1.0000x reference)
import functools

import jax
import jax.numpy as jnp
from jax import lax
from jax.experimental import pallas as pl
from jax.experimental.pallas import tpu as pltpu
from jax.experimental.pallas import tpu_sc as plsc

NC, NS = 2, 16
NW = NC * NS
D = 32
NB = 16384
NSQ = 50
CB = 1024
NBC = NB // CB
UNITS = NSQ * NBC
UPW = UNITS // NW
NG = CB // 128
DBLK = D // 8
BLK = CB // 128


def _emb_body(xt_hbm, w_hbm, out_hbm, idx_v, rows_v, t_v, sem_a, sem_b):
    wid = lax.axis_index("s") * NC + lax.axis_index("c")
    u0 = wid * UPW
    iota16 = lax.broadcasted_iota(jnp.int32, (16,), 0)

    def fire(u, buf, sem):
        s_idx = u // NBC
        b0 = (u % NBC) * CB
        pltpu.sync_copy(xt_hbm.at[s_idx, pl.ds(b0, CB)], idx_v.at[buf])
        for k in range(NG):
            pltpu.async_copy(
                w_hbm.at[idx_v.at[buf, pl.ds(k * 128, 128)]],
                rows_v.at[buf, pl.ds(k * 128, 128)],
                sem,
            )

    def drain(buf, sem):
        for k in range(NG):
            pltpu.make_async_copy(
                w_hbm.at[pl.ds(0, 128)],
                rows_v.at[buf, pl.ds(k * 128, 128)],
                sem,
            ).wait()

    fire(u0, 0, sem_a)

    def body(i, carry):
        u = u0 + i
        cbuf = lax.rem(i, 2)
        nbuf = lax.rem(i + 1, 2)

        @pl.when(i < UPW - 1)
        def _():
            @pl.when(nbuf == 0)
            def _():
                fire(u + 1, 0, sem_a)

            @pl.when(nbuf == 1)
            def _():
                fire(u + 1, 1, sem_b)

        @pl.when(cbuf == 0)
        def _():
            drain(0, sem_a)

        @pl.when(cbuf == 1)
        def _():
            drain(1, sem_b)

        bufv = jnp.full((16,), cbuf, jnp.int32)

        def bl_body(bl, c2):
            for dblk in range(DBLK):
                for dmod in range(8):
                    col = jnp.full((16,), dblk * 8 + dmod, jnp.int32)
                    for bg in range(8):
                        rowv = iota16 + (bl * 128 + bg * 16)
                        v = plsc.load_gather(rows_v, [bufv, rowv, col])
                        t_v[dblk, bl, dmod, pl.ds(bg * 16, 16)] = v
            return c2

        lax.fori_loop(0, BLK, bl_body, 0)

        s_idx = u // NBC
        bc = u % NBC
        for dblk in range(DBLK):
            pltpu.sync_copy(
                t_v.at[dblk], out_hbm.at[s_idx, dblk, pl.ds(bc * BLK, BLK)]
            )
        return carry

    lax.fori_loop(0, UPW, body, 0)


@functools.partial(
    pl.kernel,
    out_type=jax.ShapeDtypeStruct((NSQ, DBLK, NB // 128, 8, 128), jnp.float32),
    mesh=plsc.VectorSubcoreMesh(
        core_axis_name="c", subcore_axis_name="s", num_cores=NC, num_subcores=NS
    ),
    scratch_types=[
        pltpu.VMEM((2, CB), jnp.int32),
        pltpu.VMEM((2, CB, D), jnp.float32),
        pltpu.VMEM((DBLK, BLK, 8, 128), jnp.float32),
        pltpu.SemaphoreType.DMA,
        pltpu.SemaphoreType.DMA,
    ],
    compiler_params=pltpu.CompilerParams(
        use_tc_tiling_on_sc=False, needs_layout_passes=False
    ),
)
def _emb_lookup(xt_hbm, w_hbm, out_hbm, idx_v, rows_v, t_v, sem_a, sem_b):
    _emb_body(xt_hbm, w_hbm, out_hbm, idx_v, rows_v, t_v, sem_a, sem_b)


def kernel(x, weight):
    xt = x.T.astype(jnp.int32)
    o5 = _emb_lookup(xt, weight)
    return o5.transpose(2, 4, 0, 1, 3).reshape(NB, NSQ, D)


# speedup vs baseline: 1.7657x; 1.7657x over previous
import functools

import jax
import jax.numpy as jnp
from jax import lax
from jax.experimental import pallas as pl
from jax.experimental.pallas import tpu as pltpu
from jax.experimental.pallas import tpu_sc as plsc

NC, NS = 2, 16
NW = NC * NS
D = 32
NB = 16384
NSQ = 50
CB = 1024
NBC = NB // CB
UNITS = NSQ * NBC
UPW = UNITS // NW
NG = CB // 128
DBLK = D // 8
BLK = CB // 128
TP = 133
TR = DBLK * BLK * 8


def _emb_body(xt_hbm, w_hbm, out_hbm, idx_v, rows_v, t_v, sem_a, sem_b, sem_o):
    wid = lax.axis_index("s") * NC + lax.axis_index("c")
    u0 = wid * UPW
    iota16 = lax.broadcasted_iota(jnp.int32, (16,), 0)
    pat_lo = (iota16 // 8) * (BLK * 8) + (iota16 % 8)

    def fire(u, buf, sem):
        s_idx = u // NBC
        b0 = (u % NBC) * CB
        pltpu.sync_copy(xt_hbm.at[s_idx, pl.ds(b0, CB)], idx_v.at[buf])
        for k in range(NG):
            pltpu.async_copy(
                w_hbm.at[idx_v.at[buf, pl.ds(k * 128, 128)]],
                rows_v.at[buf, pl.ds(k * 128, 128)],
                sem,
            )

    def drain(buf, sem):
        for k in range(NG):
            pltpu.make_async_copy(
                w_hbm.at[pl.ds(0, 128)],
                rows_v.at[buf, pl.ds(k * 128, 128)],
                sem,
            ).wait()

    def drain_out(u):
        s_idx = u // NBC
        bc = u % NBC
        for dblk in range(DBLK):
            for bl in range(BLK):
                pltpu.make_async_copy(
                    t_v.at[pl.ds((dblk * BLK + bl) * 8, 8), pl.ds(0, 128)],
                    out_hbm.at[s_idx, dblk, bc * BLK + bl],
                    sem_o,
                ).wait()

    fire(u0, 0, sem_a)

    def body(i, carry):
        u = u0 + i
        cbuf = lax.rem(i, 2)
        nbuf = lax.rem(i + 1, 2)

        @pl.when(i < UPW - 1)
        def _():
            @pl.when(nbuf == 0)
            def _():
                fire(u + 1, 0, sem_a)

            @pl.when(nbuf == 1)
            def _():
                fire(u + 1, 1, sem_b)

        @pl.when(cbuf == 0)
        def _():
            drain(0, sem_a)

        @pl.when(cbuf == 1)
        def _():
            drain(1, sem_b)

        def bl_body(bl, c2):
            row_lo = pat_lo + bl * 8
            row_hi = row_lo + 2 * (BLK * 8)

            def j_body(jg, c3):
                for jj in range(8):
                    j = jg * 8 + jj
                    r = bl * 128 + j
                    colj = jnp.full((16,), j, jnp.int32)
                    v_lo = rows_v[cbuf, r, pl.ds(0, 16)]
                    v_hi = rows_v[cbuf, r, pl.ds(16, 16)]
                    plsc.store_scatter(t_v, [row_lo, colj], v_lo)
                    plsc.store_scatter(t_v, [row_hi, colj], v_hi)
                return c3

            lax.fori_loop(0, 16, j_body, 0)
            return c2

        lax.fori_loop(0, BLK, bl_body, 0)

        s_idx = u // NBC
        bc = u % NBC
        for dblk in range(DBLK):
            for bl in range(BLK):
                pltpu.async_copy(
                    t_v.at[pl.ds((dblk * BLK + bl) * 8, 8), pl.ds(0, 128)],
                    out_hbm.at[s_idx, dblk, bc * BLK + bl],
                    sem_o,
                )
        drain_out(u)
        return carry

    lax.fori_loop(0, UPW, body, 0)


@functools.partial(
    pl.kernel,
    out_type=jax.ShapeDtypeStruct((NSQ, DBLK, NB // 128, 8, 128), jnp.float32),
    mesh=plsc.VectorSubcoreMesh(
        core_axis_name="c", subcore_axis_name="s", num_cores=NC, num_subcores=NS
    ),
    scratch_types=[
        pltpu.VMEM((2, CB), jnp.int32),
        pltpu.VMEM((2, CB, D), jnp.float32),
        pltpu.VMEM((TR, TP), jnp.float32),
        pltpu.SemaphoreType.DMA,
        pltpu.SemaphoreType.DMA,
        pltpu.SemaphoreType.DMA,
    ],
    compiler_params=pltpu.CompilerParams(
        use_tc_tiling_on_sc=False, needs_layout_passes=False
    ),
)
def _emb_lookup(xt_hbm, w_hbm, out_hbm, idx_v, rows_v, t_v, sem_a, sem_b, sem_o):
    _emb_body(xt_hbm, w_hbm, out_hbm, idx_v, rows_v, t_v, sem_a, sem_b, sem_o)


def kernel(x, weight):
    xt = x.T.astype(jnp.int32)
    o5 = _emb_lookup(xt, weight)
    return o5.transpose(2, 4, 0, 1, 3).reshape(NB, NSQ, D)
